# baseline (speedup 1.0000x reference)
import functools

import jax
import jax.numpy as jnp
from jax import lax
from jax.experimental import pallas as pl
from jax.experimental.pallas import tpu as pltpu
from jax.experimental.pallas import tpu_sc as plsc

B, L, D = 4096, 50, 64
V = 100000
NW = 32
CB = 1024
NBC = B // CB
NCH = L * NBC
NVEC = CB // 16


@jax.jit
def _poi_gather(xT, tableT):
    mesh = plsc.VectorSubcoreMesh(core_axis_name="c", subcore_axis_name="s")

    @functools.partial(
        pl.kernel,
        out_type=jax.ShapeDtypeStruct((L * D * (B // 128), 128), jnp.float32),
        mesh=mesh,
        compiler_params=pltpu.CompilerParams(
            use_tc_tiling_on_sc=False, needs_layout_passes=False),
        scratch_types=[
            pltpu.VMEM((V,), jnp.float32),
            pltpu.VMEM((2, CB), jnp.int32),
            pltpu.VMEM((2, 8, 128), jnp.float32),
            pltpu.VMEM((2, 16), jnp.int32),
            pltpu.SemaphoreType.DMA,
            pltpu.SemaphoreType.DMA,
        ],
    )
    def k(xT_hbm, tT_hbm, out_hbm, trow, idx_c, out_c, ids_v, i_sem, o_sem):
        wid = lax.axis_index("s") * 2 + lax.axis_index("c")
        lane = lax.iota(jnp.int32, 16)

        def idx_dma(c, s):
            l, bc = c // NBC, c % NBC
            return pltpu.async_copy(
                xT_hbm.at[l, pl.ds(bc * CB, CB)], idx_c.at[s], i_sem)

        def wait_idx():
            pltpu.make_async_copy(
                xT_hbm.at[0, pl.ds(0, CB)], idx_c.at[0], i_sem).wait()

        def drain_scatter():
            pltpu.make_async_copy(
                out_c.at[0], out_hbm.at[ids_v.at[0, pl.ds(0, 8)]],
                o_sem).wait()

        def do_pass(d):
            dt, din = d // 8, d % 8
            pltpu.sync_copy(tT_hbm.at[d], trow)
            idx_dma(0, 0)
            idx_dma(1, 1)

            def phase(c, s):
                @pl.when(c > 1)
                def _drain():
                    drain_scatter()

                wait_idx()
                for i in range(NVEC):
                    iv = idx_c[s, pl.ds(i * 16, 16)]
                    vals = plsc.load_gather(trow, [iv])
                    out_c[s, i // 8, pl.ds((i % 8) * 16, 16)] = vals

                @pl.when(c < NCH - 2)
                def _next_idx():
                    idx_dma(c + 2, s)

                l, bc = c // NBC, c % NBC
                a0 = ((l * 8 + dt) * (B // 128) + bc * 8) * 8 + din
                ids_v[s] = a0 + lane * 8
                pltpu.async_copy(
                    out_c.at[s], out_hbm.at[ids_v.at[s, pl.ds(0, 8)]], o_sem)

            def body(i, carry):
                phase(2 * i, 0)
                phase(2 * i + 1, 1)
                return carry

            lax.fori_loop(0, NCH // 2, body, 0)
            drain_scatter()
            drain_scatter()

        do_pass(wid)
        do_pass(wid + NW)

    return k(xT, tableT)


def kernel(x, POI):
    o5 = _poi_gather(x.astype(jnp.int32).T, POI.T)
    return o5.reshape(L, 8, B // 128, 8, 128).transpose(
        (2, 4, 0, 1, 3)).reshape(B, L, D)


# speedup vs baseline: 1.5841x; 1.5841x over previous
import functools

import jax
import jax.numpy as jnp
from jax import lax
from jax.experimental import pallas as pl
from jax.experimental.pallas import tpu as pltpu
from jax.experimental.pallas import tpu_sc as plsc

B, L, D = 4096, 50, 64
NW = 32
RPW = B // NW
K = 8
NPH = RPW // K


@jax.jit
def _poi_gather(x, table):
    mesh = plsc.VectorSubcoreMesh(core_axis_name="c", subcore_axis_name="s")

    @functools.partial(
        pl.kernel,
        out_type=jax.ShapeDtypeStruct((B, 56, 128), jnp.float32),
        mesh=mesh,
        compiler_params=pltpu.CompilerParams(use_tc_tiling_on_sc=False),
        scratch_types=[
            pltpu.VMEM((RPW, L), jnp.int32),
            pltpu.VMEM((2, K, L, D), jnp.float32),
            pltpu.SemaphoreType.DMA,
            pltpu.SemaphoreType.DMA,
        ],
    )
    def k(x_hbm, table_hbm, out_hbm, idx_v, rows_v, g_sem, o_sem):
        wid = lax.axis_index("s") * 2 + lax.axis_index("c")
        base = wid * RPW
        pltpu.sync_copy(x_hbm.at[pl.ds(base, RPW)], idx_v)

        def fire(p, bank):
            for b in range(K):
                pltpu.async_copy(
                    table_hbm.at[idx_v.at[p * K + b]], rows_v.at[bank, b],
                    g_sem)

        def wait_gathers():
            for _ in range(K):
                pltpu.make_async_copy(
                    table_hbm.at[idx_v.at[0]], rows_v.at[0, 0], g_sem).wait()

        def puts(p, bank):
            for b in range(K):
                pltpu.async_copy(
                    rows_v.at[bank, b],
                    out_hbm.at[base + p * K + b, pl.ds(0, L), pl.ds(0, D)],
                    o_sem)

        def wait_puts():
            for _ in range(K):
                pltpu.make_async_copy(
                    rows_v.at[0, 0],
                    out_hbm.at[0, pl.ds(0, L), pl.ds(0, D)], o_sem).wait()

        fire(0, 0)

        def body(i, carry):
            for q in range(2):
                p = 2 * i + q

                @pl.when(p > 0)
                def _drain():
                    wait_puts()

                @pl.when(p < NPH - 1)
                def _prefetch():
                    fire(p + 1, 1 - q)

                wait_gathers()
                puts(p, q)
            return carry

        lax.fori_loop(0, NPH // 2, body, 0)
        wait_puts()

    return k(x, table)


def kernel(x, POI):
    big = _poi_gather(x.astype(jnp.int32), POI)
    return big[:, :L, :D]
